# Initial kernel scaffold; baseline (speedup 1.0000x reference)
import jax
import jax.numpy as jnp
from jax.experimental import pallas as pl


def kernel(x, edge_index, W1, b1, g1, be1, W2, b2, g2, be2, W3, b3):
    raise NotImplementedError("write your pallas kernel here")


# speedup vs baseline: 25.2637x; 25.2637x over previous
import functools

import jax
import jax.numpy as jnp
from jax import lax
from jax.experimental import pallas as pl
from jax.experimental.pallas import tpu as pltpu
from jax.experimental.pallas import tpu_sc as plsc

N = 10000
E = 320000
D = 128
EPS = 1e-5

NC = 2
NS = 16
NW = NC * NS
EPW = E // NW
CHUNK = 80
NCHUNK = EPW // CHUNK
NSUP = 5
SS = NCHUNK // NSUP
NPAD = 10240
STRIPE = NPAD // NS
BN = 1024
GRID = NPAD // BN


def _deg_body(dst_hbm, out_hbm, idx_v, ones_v, zbuf_v, acc_sh):
    cid = lax.axis_index("c")
    sid = lax.axis_index("s")
    wid = sid * NC + cid

    for j in range(CHUNK // 16):
        ones_v[pl.ds(16 * j, 16)] = jnp.ones((16,), jnp.float32)

    def _zfill(i, carry):
        zbuf_v[pl.ds(i * 16, 16)] = jnp.zeros((16,), jnp.float32)
        return carry

    lax.fori_loop(0, STRIPE // 16, _zfill, 0)
    pltpu.sync_copy(zbuf_v, acc_sh.at[pl.ds(sid * STRIPE, STRIPE)])
    pltpu.sync_copy(dst_hbm.at[wid], idx_v)
    plsc.subcore_barrier()

    def _body(j, carry):
        s = lax.div(j, SS)
        r = lax.rem(j, SS)
        pltpu.sync_copy(ones_v, acc_sh.at[idx_v.at[s, r]], add=True)
        return carry

    lax.fori_loop(0, NCHUNK, _body, 0)
    plsc.subcore_barrier()
    pltpu.sync_copy(acc_sh.at[pl.ds(sid * STRIPE, STRIPE)],
                    out_hbm.at[cid, pl.ds(sid * STRIPE, STRIPE)])


def _agg_body(h_hbm, src_hbm, dst_hbm, out_hbm,
              src_v, dst_v, rows_v, acc_sh, sem0, sem1):
    cid = lax.axis_index("c")
    sid = lax.axis_index("s")
    wid = sid * NC + cid

    def _zfill(i, carry):
        for k in range(D // 16):
            rows_v[0, i, pl.ds(16 * k, 16)] = jnp.zeros((16,), jnp.float32)
        return carry

    lax.fori_loop(0, CHUNK, _zfill, 0)
    for t in range(STRIPE // CHUNK):
        pltpu.sync_copy(rows_v.at[0],
                        acc_sh.at[pl.ds(sid * STRIPE + CHUNK * t, CHUNK)])
    plsc.subcore_barrier()

    buf0 = rows_v.at[0]
    buf1 = rows_v.at[1]

    def _super(s, carry):
        pltpu.sync_copy(src_hbm.at[wid, s], src_v)
        pltpu.sync_copy(dst_hbm.at[wid, s], dst_v)
        pltpu.async_copy(h_hbm.at[src_v.at[0]], buf0, sem0)

        def _pair(t, c2):
            j = 2 * t
            cp_b = pltpu.async_copy(h_hbm.at[src_v.at[j + 1]], buf1, sem1)
            pltpu.make_async_copy(h_hbm.at[src_v.at[j]], buf0, sem0).wait()
            pltpu.sync_copy(buf0, acc_sh.at[dst_v.at[j]], add=True)
            pltpu.async_copy(h_hbm.at[src_v.at[j + 2]], buf0, sem0)
            cp_b.wait()
            pltpu.sync_copy(buf1, acc_sh.at[dst_v.at[j + 1]], add=True)
            return c2

        lax.fori_loop(0, (SS - 1) // 2, _pair, 0)
        pltpu.make_async_copy(h_hbm.at[src_v.at[SS - 1]], buf0, sem0).wait()
        pltpu.sync_copy(buf0, acc_sh.at[dst_v.at[SS - 1]], add=True)
        return carry

    lax.fori_loop(0, NSUP, _super, 0)
    plsc.subcore_barrier()
    pltpu.sync_copy(acc_sh.at[pl.ds(sid * STRIPE, STRIPE)],
                    out_hbm.at[cid, pl.ds(sid * STRIPE, STRIPE)])


@functools.cache
def _sc_kernels():
    mesh = plsc.VectorSubcoreMesh(core_axis_name="c", subcore_axis_name="s",
                                  num_cores=NC, num_subcores=NS)
    deg = pl.kernel(
        _deg_body,
        out_type=jax.ShapeDtypeStruct((NC, NPAD), jnp.float32),
        mesh=mesh,
        scratch_types=[
            pltpu.VMEM((NSUP, SS, CHUNK), jnp.int32),
            pltpu.VMEM((CHUNK,), jnp.float32),
            pltpu.VMEM((STRIPE,), jnp.float32),
            pltpu.VMEM_SHARED((NPAD,), jnp.float32),
        ],
    )
    agg = pl.kernel(
        _agg_body,
        out_type=jax.ShapeDtypeStruct((NC, NPAD, D), jnp.float32),
        mesh=mesh,
        scratch_types=[
            pltpu.VMEM((SS, CHUNK), jnp.int32),
            pltpu.VMEM((SS, CHUNK), jnp.int32),
            pltpu.VMEM((2, CHUNK, D), jnp.float32),
            pltpu.VMEM_SHARED((NPAD, D), jnp.float32),
            pltpu.SemaphoreType.DMA,
            pltpu.SemaphoreType.DMA,
        ],
    )
    return deg, agg


def _dis_body(deg_ref, out_ref):
    d = deg_ref[...]
    s = d[0:1, :] + d[1:2, :] + 1.0
    col = jnp.transpose(s)
    out_ref[...] = jnp.broadcast_to(lax.rsqrt(col), (BN, D))


def _dis_full(degp):
    return pl.pallas_call(
        _dis_body,
        grid=(GRID,),
        in_specs=[pl.BlockSpec((NC, BN), lambda i: (0, i))],
        out_specs=pl.BlockSpec((BN, D), lambda i: (i, 0)),
        out_shape=jax.ShapeDtypeStruct((NPAD, D), jnp.float32),
    )(degp)


def _pre_body(x_ref, w_ref, dis_ref, out_ref):
    out_ref[...] = dis_ref[...] * jnp.dot(
        x_ref[...], w_ref[...], preferred_element_type=jnp.float32)


def _tc_pre(x, w, dis):
    return pl.pallas_call(
        _pre_body,
        grid=(GRID,),
        in_specs=[
            pl.BlockSpec((BN, D), lambda i: (i, 0)),
            pl.BlockSpec((D, D), lambda i: (0, 0)),
            pl.BlockSpec((BN, D), lambda i: (i, 0)),
        ],
        out_specs=pl.BlockSpec((BN, D), lambda i: (i, 0)),
        out_shape=jax.ShapeDtypeStruct((NPAD, D), jnp.float32),
    )(x, w, dis)


def _mid_body(p_ref, hp_ref, dis_ref, b_ref, g_ref, be_ref, w_ref, out_ref):
    dis = dis_ref[...]
    a = dis * (p_ref[0] + p_ref[1] + hp_ref[...]) + b_ref[...]
    mu = jnp.mean(a, axis=-1, keepdims=True)
    var = jnp.mean((a - mu) ** 2, axis=-1, keepdims=True)
    r = jnp.maximum((a - mu) * lax.rsqrt(var + EPS) * g_ref[...] + be_ref[...],
                    0.0)
    out_ref[...] = dis * jnp.dot(r, w_ref[...],
                                 preferred_element_type=jnp.float32)


def _tc_mid(p, hp, dis, b, g, be, w):
    return pl.pallas_call(
        _mid_body,
        grid=(GRID,),
        in_specs=[
            pl.BlockSpec((NC, BN, D), lambda i: (0, i, 0)),
            pl.BlockSpec((BN, D), lambda i: (i, 0)),
            pl.BlockSpec((BN, D), lambda i: (i, 0)),
            pl.BlockSpec((1, D), lambda i: (0, 0)),
            pl.BlockSpec((1, D), lambda i: (0, 0)),
            pl.BlockSpec((1, D), lambda i: (0, 0)),
            pl.BlockSpec((D, D), lambda i: (0, 0)),
        ],
        out_specs=pl.BlockSpec((BN, D), lambda i: (i, 0)),
        out_shape=jax.ShapeDtypeStruct((NPAD, D), jnp.float32),
    )(p, hp, dis, b.reshape(1, D), g.reshape(1, D), be.reshape(1, D), w)


def _fin_body(p_ref, hp_ref, dis_ref, b_ref, out_ref):
    out_ref[...] = (dis_ref[...] * (p_ref[0] + p_ref[1] + hp_ref[...])
                    + b_ref[...])


def _tc_fin(p, hp, dis, b):
    return pl.pallas_call(
        _fin_body,
        grid=(GRID,),
        in_specs=[
            pl.BlockSpec((NC, BN, D), lambda i: (0, i, 0)),
            pl.BlockSpec((BN, D), lambda i: (i, 0)),
            pl.BlockSpec((BN, D), lambda i: (i, 0)),
            pl.BlockSpec((1, D), lambda i: (0, 0)),
        ],
        out_specs=pl.BlockSpec((BN, D), lambda i: (i, 0)),
        out_shape=jax.ShapeDtypeStruct((NPAD, D), jnp.float32),
    )(p, hp, dis, b.reshape(1, D))


def kernel(x, edge_index, W1, b1, g1, be1, W2, b2, g2, be2, W3, b3):
    src = edge_index[0].reshape(NW, NSUP, SS, CHUNK)
    dst = edge_index[1].reshape(NW, NSUP, SS, CHUNK)
    xp = jnp.pad(x, ((0, NPAD - N), (0, 0)))

    deg_kernel, agg_kernel = _sc_kernels()
    degp = deg_kernel(dst)
    dis = _dis_full(degp)

    hp1 = _tc_pre(xp, W1, dis)
    p1 = agg_kernel(hp1, src, dst)
    hp2 = _tc_mid(p1, hp1, dis, b1, g1, be1, W2)
    p2 = agg_kernel(hp2, src, dst)
    hp3 = _tc_mid(p2, hp2, dis, b2, g2, be2, W3)
    p3 = agg_kernel(hp3, src, dst)
    out = _tc_fin(p3, hp3, dis, b3)
    return out[:N]


# speedup vs baseline: 28.3606x; 1.1226x over previous
import functools

import jax
import jax.numpy as jnp
from jax import lax
from jax.experimental import pallas as pl
from jax.experimental.pallas import tpu as pltpu
from jax.experimental.pallas import tpu_sc as plsc

N = 10000
E = 320000
D = 128
EPS = 1e-5

NC = 2
NS = 16
NW = NC * NS
EPW = E // NW
CHUNK = 80
NCHUNK = EPW // CHUNK
NSUP = 5
SS = NCHUNK // NSUP
NPAD = 10240
STRIPE = NPAD // NS
BN = 1024
GRID = NPAD // BN


def _deg_body(dst_hbm, out_hbm, idx_v, ones_v, zbuf_v, acc_sh):
    cid = lax.axis_index("c")
    sid = lax.axis_index("s")
    wid = sid * NC + cid

    for j in range(CHUNK // 16):
        ones_v[pl.ds(16 * j, 16)] = jnp.ones((16,), jnp.float32)

    def _zfill(i, carry):
        zbuf_v[pl.ds(i * 16, 16)] = jnp.zeros((16,), jnp.float32)
        return carry

    lax.fori_loop(0, STRIPE // 16, _zfill, 0)
    pltpu.sync_copy(zbuf_v, acc_sh.at[pl.ds(sid * STRIPE, STRIPE)])
    pltpu.sync_copy(dst_hbm.at[wid], idx_v)
    plsc.subcore_barrier()

    def _body(j, carry):
        s = lax.div(j, SS)
        r = lax.rem(j, SS)
        pltpu.sync_copy(ones_v, acc_sh.at[idx_v.at[s, r]], add=True)
        return carry

    lax.fori_loop(0, NCHUNK, _body, 0)
    plsc.subcore_barrier()
    pltpu.sync_copy(acc_sh.at[pl.ds(sid * STRIPE, STRIPE)],
                    out_hbm.at[cid, pl.ds(sid * STRIPE, STRIPE)])


def _agg_body(h_hbm, src_hbm, dst_hbm, out_hbm,
              src_v, dst_v, rows_v, acc_sh, semg, sems):
    cid = lax.axis_index("c")
    sid = lax.axis_index("s")
    wid = sid * NC + cid

    def _zfill(i, carry):
        for k in range(D // 16):
            rows_v[0, i, pl.ds(16 * k, 16)] = jnp.zeros((16,), jnp.float32)
        return carry

    lax.fori_loop(0, CHUNK, _zfill, 0)
    for t in range(STRIPE // CHUNK):
        pltpu.async_copy(rows_v.at[0],
                         acc_sh.at[pl.ds(sid * STRIPE + CHUNK * t, CHUNK)],
                         semg.at[0])
    for t in range(STRIPE // CHUNK):
        pltpu.make_async_copy(
            rows_v.at[0],
            acc_sh.at[pl.ds(sid * STRIPE + CHUNK * t, CHUNK)],
            semg.at[0]).wait()
    plsc.subcore_barrier()

    def _gather(j, b):
        return pltpu.async_copy(h_hbm.at[src_v.at[j]], rows_v.at[b],
                                semg.at[b])

    def _scat_start(j, b):
        pltpu.async_copy(rows_v.at[b], acc_sh.at[dst_v.at[j]], sems.at[b],
                         add=True)

    def _scat_wait(j, b):
        pltpu.make_async_copy(rows_v.at[b], acc_sh.at[dst_v.at[j]],
                              sems.at[b]).wait()

    def _super(s, carry):
        pltpu.sync_copy(src_hbm.at[wid, s], src_v)
        pltpu.sync_copy(dst_hbm.at[wid, s], dst_v)
        _gather(0, 0)
        _gather(1, 1)
        for j in range(SS):
            b = j % 3
            pltpu.make_async_copy(h_hbm.at[src_v.at[j]], rows_v.at[b],
                                  semg.at[b]).wait()
            _scat_start(j, b)
            if j + 2 < SS:
                bn = (j + 2) % 3
                if j >= 1:
                    _scat_wait(j - 1, bn)
                _gather(j + 2, bn)
        for j in range(SS - 3, SS):
            _scat_wait(j, j % 3)
        return carry

    lax.fori_loop(0, NSUP, _super, 0)
    plsc.subcore_barrier()
    pltpu.sync_copy(acc_sh.at[pl.ds(sid * STRIPE, STRIPE)],
                    out_hbm.at[cid, pl.ds(sid * STRIPE, STRIPE)])


@functools.cache
def _sc_kernels():
    mesh = plsc.VectorSubcoreMesh(core_axis_name="c", subcore_axis_name="s",
                                  num_cores=NC, num_subcores=NS)
    deg = pl.kernel(
        _deg_body,
        out_type=jax.ShapeDtypeStruct((NC, NPAD), jnp.float32),
        mesh=mesh,
        scratch_types=[
            pltpu.VMEM((NSUP, SS, CHUNK), jnp.int32),
            pltpu.VMEM((CHUNK,), jnp.float32),
            pltpu.VMEM((STRIPE,), jnp.float32),
            pltpu.VMEM_SHARED((NPAD,), jnp.float32),
        ],
    )
    agg = pl.kernel(
        _agg_body,
        out_type=jax.ShapeDtypeStruct((NC, NPAD, D), jnp.float32),
        mesh=mesh,
        scratch_types=[
            pltpu.VMEM((SS, CHUNK), jnp.int32),
            pltpu.VMEM((SS, CHUNK), jnp.int32),
            pltpu.VMEM((3, CHUNK, D), jnp.float32),
            pltpu.VMEM_SHARED((NPAD, D), jnp.float32),
            pltpu.SemaphoreType.DMA((3,)),
            pltpu.SemaphoreType.DMA((3,)),
        ],
    )
    return deg, agg


def _dis_body(deg_ref, out_ref):
    d = deg_ref[...]
    s = d[0:1, :] + d[1:2, :] + 1.0
    col = jnp.transpose(s)
    out_ref[...] = jnp.broadcast_to(lax.rsqrt(col), (BN, D))


def _dis_full(degp):
    return pl.pallas_call(
        _dis_body,
        grid=(GRID,),
        in_specs=[pl.BlockSpec((NC, BN), lambda i: (0, i))],
        out_specs=pl.BlockSpec((BN, D), lambda i: (i, 0)),
        out_shape=jax.ShapeDtypeStruct((NPAD, D), jnp.float32),
    )(degp)


def _pre_body(x_ref, w_ref, dis_ref, out_ref):
    out_ref[...] = dis_ref[...] * jnp.dot(
        x_ref[...], w_ref[...], preferred_element_type=jnp.float32)


def _tc_pre(x, w, dis):
    return pl.pallas_call(
        _pre_body,
        grid=(GRID,),
        in_specs=[
            pl.BlockSpec((BN, D), lambda i: (i, 0)),
            pl.BlockSpec((D, D), lambda i: (0, 0)),
            pl.BlockSpec((BN, D), lambda i: (i, 0)),
        ],
        out_specs=pl.BlockSpec((BN, D), lambda i: (i, 0)),
        out_shape=jax.ShapeDtypeStruct((NPAD, D), jnp.float32),
    )(x, w, dis)


def _mid_body(p_ref, hp_ref, dis_ref, b_ref, g_ref, be_ref, w_ref, out_ref):
    dis = dis_ref[...]
    a = dis * (p_ref[0] + p_ref[1] + hp_ref[...]) + b_ref[...]
    mu = jnp.mean(a, axis=-1, keepdims=True)
    var = jnp.mean((a - mu) ** 2, axis=-1, keepdims=True)
    r = jnp.maximum((a - mu) * lax.rsqrt(var + EPS) * g_ref[...] + be_ref[...],
                    0.0)
    out_ref[...] = dis * jnp.dot(r, w_ref[...],
                                 preferred_element_type=jnp.float32)


def _tc_mid(p, hp, dis, b, g, be, w):
    return pl.pallas_call(
        _mid_body,
        grid=(GRID,),
        in_specs=[
            pl.BlockSpec((NC, BN, D), lambda i: (0, i, 0)),
            pl.BlockSpec((BN, D), lambda i: (i, 0)),
            pl.BlockSpec((BN, D), lambda i: (i, 0)),
            pl.BlockSpec((1, D), lambda i: (0, 0)),
            pl.BlockSpec((1, D), lambda i: (0, 0)),
            pl.BlockSpec((1, D), lambda i: (0, 0)),
            pl.BlockSpec((D, D), lambda i: (0, 0)),
        ],
        out_specs=pl.BlockSpec((BN, D), lambda i: (i, 0)),
        out_shape=jax.ShapeDtypeStruct((NPAD, D), jnp.float32),
    )(p, hp, dis, b.reshape(1, D), g.reshape(1, D), be.reshape(1, D), w)


def _fin_body(p_ref, hp_ref, dis_ref, b_ref, out_ref):
    out_ref[...] = (dis_ref[...] * (p_ref[0] + p_ref[1] + hp_ref[...])
                    + b_ref[...])


def _tc_fin(p, hp, dis, b):
    return pl.pallas_call(
        _fin_body,
        grid=(GRID,),
        in_specs=[
            pl.BlockSpec((NC, BN, D), lambda i: (0, i, 0)),
            pl.BlockSpec((BN, D), lambda i: (i, 0)),
            pl.BlockSpec((BN, D), lambda i: (i, 0)),
            pl.BlockSpec((1, D), lambda i: (0, 0)),
        ],
        out_specs=pl.BlockSpec((BN, D), lambda i: (i, 0)),
        out_shape=jax.ShapeDtypeStruct((NPAD, D), jnp.float32),
    )(p, hp, dis, b.reshape(1, D))


def kernel(x, edge_index, W1, b1, g1, be1, W2, b2, g2, be2, W3, b3):
    src = edge_index[0].reshape(NW, NSUP, SS, CHUNK)
    dst = edge_index[1].reshape(NW, NSUP, SS, CHUNK)
    xp = jnp.pad(x, ((0, NPAD - N), (0, 0)))

    deg_kernel, agg_kernel = _sc_kernels()
    degp = deg_kernel(dst)
    dis = _dis_full(degp)

    hp1 = _tc_pre(xp, W1, dis)
    p1 = agg_kernel(hp1, src, dst)
    hp2 = _tc_mid(p1, hp1, dis, b1, g1, be1, W2)
    p2 = agg_kernel(hp2, src, dst)
    hp3 = _tc_mid(p2, hp2, dis, b2, g2, be2, W3)
    p3 = agg_kernel(hp3, src, dst)
    out = _tc_fin(p3, hp3, dis, b3)
    return out[:N]


# speedup vs baseline: 28.7798x; 1.0148x over previous
import functools

import jax
import jax.numpy as jnp
from jax import lax
from jax.experimental import pallas as pl
from jax.experimental.pallas import tpu as pltpu
from jax.experimental.pallas import tpu_sc as plsc

N = 10000
E = 320000
D = 128
EPS = 1e-5

NC = 2
NS = 16
NW = NC * NS
EPW = E // NW
CHUNK = 80
NCHUNK = EPW // CHUNK
NSUP = 5
SS = NCHUNK // NSUP
NPAD = 10240
STRIPE = NPAD // NS
ASTRIPE = N // NS
BN = 1024
GRID = NPAD // BN


def _deg_body(dst_hbm, out_hbm, idx_v, ones_v, zbuf_v, acc_sh, dsem):
    cid = lax.axis_index("c")
    sid = lax.axis_index("s")
    wid = sid * NC + cid

    for j in range(CHUNK // 16):
        ones_v[pl.ds(16 * j, 16)] = jnp.ones((16,), jnp.float32)

    def _zfill(i, carry):
        zbuf_v[pl.ds(i * 16, 16)] = jnp.zeros((16,), jnp.float32)
        return carry

    lax.fori_loop(0, STRIPE // 16, _zfill, 0)
    pltpu.sync_copy(zbuf_v, acc_sh.at[pl.ds(sid * STRIPE, STRIPE)])
    pltpu.sync_copy(dst_hbm.at[wid], idx_v)
    plsc.subcore_barrier()

    def _wave(s, carry):
        def _fire(r, c2):
            pltpu.async_copy(ones_v, acc_sh.at[idx_v.at[s, r]], dsem,
                             add=True)
            return c2

        def _drain(r, c2):
            pltpu.make_async_copy(ones_v, acc_sh.at[idx_v.at[s, r]],
                                  dsem).wait()
            return c2

        lax.fori_loop(0, SS, _fire, 0)
        lax.fori_loop(0, SS, _drain, 0)
        return carry

    lax.fori_loop(0, NSUP, _wave, 0)
    plsc.subcore_barrier()
    pltpu.sync_copy(acc_sh.at[pl.ds(sid * STRIPE, STRIPE)],
                    out_hbm.at[cid, pl.ds(sid * STRIPE, STRIPE)])


def _agg_body(h_hbm, src_hbm, dst_hbm, out_hbm,
              src_v, dst_v, rows_v, acc_sh, semg, sems):
    cid = lax.axis_index("c")
    sid = lax.axis_index("s")
    wid = sid * NC + cid

    def _zfill(i, carry):
        for k in range(D // 16):
            rows_v[0, i, pl.ds(16 * k, 16)] = jnp.zeros((16,), jnp.float32)
        return carry

    lax.fori_loop(0, CHUNK, _zfill, 0)
    for t in range(STRIPE // CHUNK):
        pltpu.async_copy(rows_v.at[0],
                         acc_sh.at[pl.ds(sid * STRIPE + CHUNK * t, CHUNK)],
                         semg.at[0])
    for t in range(STRIPE // CHUNK):
        pltpu.make_async_copy(
            rows_v.at[0],
            acc_sh.at[pl.ds(sid * STRIPE + CHUNK * t, CHUNK)],
            semg.at[0]).wait()
    plsc.subcore_barrier()

    def _super(s, carry):
        pltpu.sync_copy(src_hbm.at[wid, s], src_v)
        pltpu.sync_copy(dst_hbm.at[wid, s], dst_v)

        def _gather(j, b):
            pltpu.async_copy(h_hbm.at[src_v.at[j]], rows_v.at[b],
                             semg.at[b])

        def _scat_wait(j, b):
            pltpu.make_async_copy(rows_v.at[b], acc_sh.at[dst_v.at[j]],
                                  sems.at[b]).wait()

        _gather(0, 0)
        _gather(1, 1)
        for j in range(SS):
            b = j % 3
            pltpu.make_async_copy(h_hbm.at[src_v.at[j]], rows_v.at[b],
                                  semg.at[b]).wait()
            pltpu.async_copy(rows_v.at[b], acc_sh.at[dst_v.at[j]],
                             sems.at[b], add=True)
            if j + 2 < SS:
                bn = (j + 2) % 3
                if j >= 1:
                    _scat_wait(j - 1, bn)
                _gather(j + 2, bn)

        for j in range(SS - 3, SS):
            _scat_wait(j, j % 3)
        return carry

    lax.fori_loop(0, NSUP, _super, 0)
    plsc.subcore_barrier()
    pltpu.sync_copy(acc_sh.at[pl.ds(sid * STRIPE, STRIPE)],
                    out_hbm.at[cid, pl.ds(sid * STRIPE, STRIPE)])


@functools.cache
def _sc_kernels():
    mesh = plsc.VectorSubcoreMesh(core_axis_name="c", subcore_axis_name="s",
                                  num_cores=NC, num_subcores=NS)
    deg = pl.kernel(
        _deg_body,
        out_type=jax.ShapeDtypeStruct((NC, NPAD), jnp.float32),
        mesh=mesh,
        scratch_types=[
            pltpu.VMEM((NSUP, SS, CHUNK), jnp.int32),
            pltpu.VMEM((CHUNK,), jnp.float32),
            pltpu.VMEM((STRIPE,), jnp.float32),
            pltpu.VMEM_SHARED((NPAD,), jnp.float32),
            pltpu.SemaphoreType.DMA,
        ],
    )
    agg = pl.kernel(
        _agg_body,
        out_type=jax.ShapeDtypeStruct((NC, NPAD, D), jnp.float32),
        mesh=mesh,
        scratch_types=[
            pltpu.VMEM((SS, CHUNK), jnp.int32),
            pltpu.VMEM((SS, CHUNK), jnp.int32),
            pltpu.VMEM((3, CHUNK, D), jnp.float32),
            pltpu.VMEM_SHARED((NPAD, D), jnp.float32),
            pltpu.SemaphoreType.DMA((3,)),
            pltpu.SemaphoreType.DMA((3,)),
        ],
    )
    return deg, agg


def _mm_body(x_ref, w_ref, out_ref):
    out_ref[...] = jnp.dot(x_ref[...], w_ref[...],
                           preferred_element_type=jnp.float32)


def _tc_mm(x, w):
    return pl.pallas_call(
        _mm_body,
        grid=(GRID,),
        in_specs=[
            pl.BlockSpec((BN, D), lambda i: (i, 0)),
            pl.BlockSpec((D, D), lambda i: (0, 0)),
        ],
        out_specs=pl.BlockSpec((BN, D), lambda i: (i, 0)),
        out_shape=jax.ShapeDtypeStruct((NPAD, D), jnp.float32),
    )(x, w)


def _dis_body(deg_ref, h_ref, dis_ref, hp_ref):
    d = deg_ref[...]
    s = d[0:1, :] + d[1:2, :] + 1.0
    col = jnp.transpose(s)
    dis = jnp.broadcast_to(lax.rsqrt(col), (BN, D))
    dis_ref[...] = dis
    hp_ref[...] = dis * h_ref[...]


def _dis_hp(degp, h1):
    return pl.pallas_call(
        _dis_body,
        grid=(GRID,),
        in_specs=[
            pl.BlockSpec((NC, BN), lambda i: (0, i)),
            pl.BlockSpec((BN, D), lambda i: (i, 0)),
        ],
        out_specs=[
            pl.BlockSpec((BN, D), lambda i: (i, 0)),
            pl.BlockSpec((BN, D), lambda i: (i, 0)),
        ],
        out_shape=[
            jax.ShapeDtypeStruct((NPAD, D), jnp.float32),
            jax.ShapeDtypeStruct((NPAD, D), jnp.float32),
        ],
    )(degp, h1)


def _mid_body(p_ref, hp_ref, dis_ref, b_ref, g_ref, be_ref, w_ref, out_ref):
    dis = dis_ref[...]
    a = dis * (p_ref[0] + p_ref[1] + hp_ref[...]) + b_ref[...]
    mu = jnp.mean(a, axis=-1, keepdims=True)
    var = jnp.mean((a - mu) ** 2, axis=-1, keepdims=True)
    r = jnp.maximum((a - mu) * lax.rsqrt(var + EPS) * g_ref[...] + be_ref[...],
                    0.0)
    out_ref[...] = dis * jnp.dot(r, w_ref[...],
                                 preferred_element_type=jnp.float32)


def _tc_mid(p, hp, dis, b, g, be, w):
    return pl.pallas_call(
        _mid_body,
        grid=(GRID,),
        in_specs=[
            pl.BlockSpec((NC, BN, D), lambda i: (0, i, 0)),
            pl.BlockSpec((BN, D), lambda i: (i, 0)),
            pl.BlockSpec((BN, D), lambda i: (i, 0)),
            pl.BlockSpec((1, D), lambda i: (0, 0)),
            pl.BlockSpec((1, D), lambda i: (0, 0)),
            pl.BlockSpec((1, D), lambda i: (0, 0)),
            pl.BlockSpec((D, D), lambda i: (0, 0)),
        ],
        out_specs=pl.BlockSpec((BN, D), lambda i: (i, 0)),
        out_shape=jax.ShapeDtypeStruct((NPAD, D), jnp.float32),
    )(p, hp, dis, b.reshape(1, D), g.reshape(1, D), be.reshape(1, D), w)


def _fin_body(p_ref, hp_ref, dis_ref, b_ref, out_ref):
    out_ref[...] = (dis_ref[...] * (p_ref[0] + p_ref[1] + hp_ref[...])
                    + b_ref[...])


def _tc_fin(p, hp, dis, b):
    return pl.pallas_call(
        _fin_body,
        grid=(GRID,),
        in_specs=[
            pl.BlockSpec((NC, BN, D), lambda i: (0, i, 0)),
            pl.BlockSpec((BN, D), lambda i: (i, 0)),
            pl.BlockSpec((BN, D), lambda i: (i, 0)),
            pl.BlockSpec((1, D), lambda i: (0, 0)),
        ],
        out_specs=pl.BlockSpec((BN, D), lambda i: (i, 0)),
        out_shape=jax.ShapeDtypeStruct((NPAD, D), jnp.float32),
    )(p, hp, dis, b.reshape(1, D))


def kernel(x, edge_index, W1, b1, g1, be1, W2, b2, g2, be2, W3, b3):
    src = edge_index[0].reshape(NW, NSUP, SS, CHUNK)
    dst = edge_index[1].reshape(NW, NSUP, SS, CHUNK)
    xp = jnp.pad(x, ((0, NPAD - N), (0, 0)))

    deg_kernel, agg_kernel = _sc_kernels()
    h1 = _tc_mm(xp, W1)
    degp = deg_kernel(dst)
    dis, hp1 = _dis_hp(degp, h1)
    p1 = agg_kernel(hp1, src, dst)
    hp2 = _tc_mid(p1, hp1, dis, b1, g1, be1, W2)
    p2 = agg_kernel(hp2, src, dst)
    hp3 = _tc_mid(p2, hp2, dis, b2, g2, be2, W3)
    p3 = agg_kernel(hp3, src, dst)
    out = _tc_fin(p3, hp3, dis, b3)
    return out[:N]


# speedup vs baseline: 32.0770x; 1.1146x over previous
import functools

import jax
import jax.numpy as jnp
from jax import lax
from jax.experimental import pallas as pl
from jax.experimental.pallas import tpu as pltpu
from jax.experimental.pallas import tpu_sc as plsc

N = 10000
E = 320000
D = 128
EPS = 1e-5

NC = 2
NS = 16
NW = NC * NS
EPW = E // NW
CHUNK = 80
NCHUNK = EPW // CHUNK
ISLOT = 6
UNROLL = 6
NPAD = 10240
STRIPE = NPAD // NS
ASTRIPE = N // NS
BN = 1024
GRID = NPAD // BN


def _deg_body(dst_hbm, out_hbm, idx_v, ones_v, zbuf_v, acc_sh, dsem):
    cid = lax.axis_index("c")
    sid = lax.axis_index("s")
    wid = sid * NC + cid

    for j in range(CHUNK // 16):
        ones_v[pl.ds(16 * j, 16)] = jnp.ones((16,), jnp.float32)

    def _zfill(i, carry):
        zbuf_v[pl.ds(i * 16, 16)] = jnp.zeros((16,), jnp.float32)
        return carry

    lax.fori_loop(0, STRIPE // 16, _zfill, 0)
    pltpu.sync_copy(zbuf_v, acc_sh.at[pl.ds(sid * STRIPE, STRIPE)])
    pltpu.sync_copy(dst_hbm.at[pl.ds(wid * NCHUNK, NCHUNK)], idx_v)
    plsc.subcore_barrier()

    pltpu.async_copy(ones_v, acc_sh.at[idx_v.at[0, 0]], dsem, add=True)
    pltpu.async_copy(ones_v, acc_sh.at[idx_v.at[1, 0]], dsem, add=True)

    def _body(j, carry):
        pltpu.make_async_copy(ones_v, acc_sh.at[idx_v.at[j, 0]],
                              dsem).wait()
        pltpu.async_copy(ones_v, acc_sh.at[idx_v.at[j + 2, 0]], dsem,
                         add=True)
        return carry

    lax.fori_loop(0, NCHUNK - 2, _body, 0)
    for j in (NCHUNK - 2, NCHUNK - 1):
        pltpu.make_async_copy(ones_v, acc_sh.at[idx_v.at[j, 0]],
                              dsem).wait()
    plsc.subcore_barrier()
    pltpu.sync_copy(acc_sh.at[pl.ds(sid * STRIPE, STRIPE)],
                    out_hbm.at[cid, pl.ds(sid * STRIPE, STRIPE)])


def _agg_body(h_hbm, src_hbm, dst_hbm, out_hbm,
              srcb_v, dstb_v, rows_v, acc_sh, semg, sems, semi):
    cid = lax.axis_index("c")
    sid = lax.axis_index("s")
    wid = sid * NC + cid

    def _zfill(i, carry):
        for k in range(D // 16):
            rows_v[0, i, pl.ds(16 * k, 16)] = jnp.zeros((16,), jnp.float32)
        return carry

    lax.fori_loop(0, CHUNK, _zfill, 0)
    for t in range(STRIPE // CHUNK):
        pltpu.async_copy(rows_v.at[0],
                         acc_sh.at[pl.ds(sid * STRIPE + CHUNK * t, CHUNK)],
                         semg.at[0])
    for t in range(STRIPE // CHUNK):
        pltpu.make_async_copy(
            rows_v.at[0],
            acc_sh.at[pl.ds(sid * STRIPE + CHUNK * t, CHUNK)],
            semg.at[0]).wait()
    plsc.subcore_barrier()

    def _ifetch(j, c):
        pltpu.async_copy(src_hbm.at[wid * NCHUNK + j], srcb_v.at[c % ISLOT],
                         semi)
        pltpu.async_copy(dst_hbm.at[wid * NCHUNK + j], dstb_v.at[c % ISLOT],
                         semi)

    def _iwait(j, c):
        pltpu.make_async_copy(src_hbm.at[wid * NCHUNK + j],
                              srcb_v.at[c % ISLOT], semi).wait()
        pltpu.make_async_copy(dst_hbm.at[wid * NCHUNK + j],
                              dstb_v.at[c % ISLOT], semi).wait()

    def _gather(c, b):
        pltpu.async_copy(h_hbm.at[srcb_v.at[c % ISLOT, 0]], rows_v.at[b],
                         semg.at[b])

    def _gwait(c, b):
        pltpu.make_async_copy(h_hbm.at[srcb_v.at[c % ISLOT, 0]],
                              rows_v.at[b], semg.at[b]).wait()

    def _scat(c, b):
        pltpu.async_copy(rows_v.at[b], acc_sh.at[dstb_v.at[c % ISLOT, 0]],
                         sems.at[b], add=True)

    def _swait(c, b):
        pltpu.make_async_copy(rows_v.at[b],
                              acc_sh.at[dstb_v.at[c % ISLOT, 0]],
                              sems.at[b]).wait()

    def _step(j, c, first=False, ng=True, nf=True):
        b = c % 3
        bn = (c + 2) % 3
        if not first:
            _swait(c - 1, bn)
        if ng:
            _iwait(j + 2, c + 2)
            _gather(c + 2, bn)
        if nf:
            _ifetch(j + 4, c + 4)
        _gwait(c, b)
        _scat(c, b)

    for j in range(4):
        _ifetch(j, j)
    _iwait(0, 0)
    _gather(0, 0)
    _iwait(1, 1)
    _gather(1, 1)

    _step(0, 0, first=True)
    for c in range(1, UNROLL):
        _step(c, c)

    def _group(t, carry):
        for c in range(UNROLL):
            _step(UNROLL * t + c, c)
        return carry

    lax.fori_loop(1, (NCHUNK - 5) // UNROLL, _group, 0)
    for j in range(NCHUNK - 5, NCHUNK):
        _step(j, j % UNROLL, ng=(j + 2 < NCHUNK), nf=(j + 4 < NCHUNK))
    _swait((NCHUNK - 1) % UNROLL, (NCHUNK - 1) % 3)
    plsc.subcore_barrier()
    pltpu.sync_copy(acc_sh.at[pl.ds(sid * STRIPE, STRIPE)],
                    out_hbm.at[cid, pl.ds(sid * STRIPE, STRIPE)])


@functools.cache
def _sc_kernels():
    mesh = plsc.VectorSubcoreMesh(core_axis_name="c", subcore_axis_name="s",
                                  num_cores=NC, num_subcores=NS)
    deg = pl.kernel(
        _deg_body,
        out_type=jax.ShapeDtypeStruct((NC, NPAD), jnp.float32),
        mesh=mesh,
        scratch_types=[
            pltpu.VMEM((NCHUNK, 1, CHUNK), jnp.int32),
            pltpu.VMEM((CHUNK,), jnp.float32),
            pltpu.VMEM((STRIPE,), jnp.float32),
            pltpu.VMEM_SHARED((NPAD,), jnp.float32),
            pltpu.SemaphoreType.DMA,
        ],
    )
    agg = pl.kernel(
        _agg_body,
        out_type=jax.ShapeDtypeStruct((NC, NPAD, D), jnp.float32),
        mesh=mesh,
        scratch_types=[
            pltpu.VMEM((ISLOT, 1, CHUNK), jnp.int32),
            pltpu.VMEM((ISLOT, 1, CHUNK), jnp.int32),
            pltpu.VMEM((3, CHUNK, D), jnp.float32),
            pltpu.VMEM_SHARED((NPAD, D), jnp.float32),
            pltpu.SemaphoreType.DMA((3,)),
            pltpu.SemaphoreType.DMA((3,)),
            pltpu.SemaphoreType.DMA,
        ],
    )
    return deg, agg


def _mm_body(x_ref, w_ref, out_ref):
    out_ref[...] = jnp.dot(x_ref[...], w_ref[...],
                           preferred_element_type=jnp.float32)


def _tc_mm(x, w):
    return pl.pallas_call(
        _mm_body,
        grid=(GRID,),
        in_specs=[
            pl.BlockSpec((BN, D), lambda i: (i, 0)),
            pl.BlockSpec((D, D), lambda i: (0, 0)),
        ],
        out_specs=pl.BlockSpec((BN, D), lambda i: (i, 0)),
        out_shape=jax.ShapeDtypeStruct((NPAD, D), jnp.float32),
    )(x, w)


def _dis_body(deg_ref, h_ref, dis_ref, hp_ref):
    d = deg_ref[...]
    s = d[0:1, :] + d[1:2, :] + 1.0
    col = jnp.transpose(s)
    dis = jnp.broadcast_to(lax.rsqrt(col), (BN, D))
    dis_ref[...] = dis
    hp_ref[...] = dis * h_ref[...]


def _dis_hp(degp, h1):
    return pl.pallas_call(
        _dis_body,
        grid=(GRID,),
        in_specs=[
            pl.BlockSpec((NC, BN), lambda i: (0, i)),
            pl.BlockSpec((BN, D), lambda i: (i, 0)),
        ],
        out_specs=[
            pl.BlockSpec((BN, D), lambda i: (i, 0)),
            pl.BlockSpec((BN, D), lambda i: (i, 0)),
        ],
        out_shape=[
            jax.ShapeDtypeStruct((NPAD, D), jnp.float32),
            jax.ShapeDtypeStruct((NPAD, D), jnp.float32),
        ],
    )(degp, h1)


def _mid_body(p_ref, hp_ref, dis_ref, b_ref, g_ref, be_ref, w_ref, out_ref):
    dis = dis_ref[...]
    a = dis * (p_ref[0] + p_ref[1] + hp_ref[...]) + b_ref[...]
    mu = jnp.mean(a, axis=-1, keepdims=True)
    var = jnp.mean((a - mu) ** 2, axis=-1, keepdims=True)
    r = jnp.maximum((a - mu) * lax.rsqrt(var + EPS) * g_ref[...] + be_ref[...],
                    0.0)
    out_ref[...] = dis * jnp.dot(r, w_ref[...],
                                 preferred_element_type=jnp.float32)


def _tc_mid(p, hp, dis, b, g, be, w):
    return pl.pallas_call(
        _mid_body,
        grid=(GRID,),
        in_specs=[
            pl.BlockSpec((NC, BN, D), lambda i: (0, i, 0)),
            pl.BlockSpec((BN, D), lambda i: (i, 0)),
            pl.BlockSpec((BN, D), lambda i: (i, 0)),
            pl.BlockSpec((1, D), lambda i: (0, 0)),
            pl.BlockSpec((1, D), lambda i: (0, 0)),
            pl.BlockSpec((1, D), lambda i: (0, 0)),
            pl.BlockSpec((D, D), lambda i: (0, 0)),
        ],
        out_specs=pl.BlockSpec((BN, D), lambda i: (i, 0)),
        out_shape=jax.ShapeDtypeStruct((NPAD, D), jnp.float32),
    )(p, hp, dis, b.reshape(1, D), g.reshape(1, D), be.reshape(1, D), w)


def _fin_body(p_ref, hp_ref, dis_ref, b_ref, out_ref):
    out_ref[...] = (dis_ref[...] * (p_ref[0] + p_ref[1] + hp_ref[...])
                    + b_ref[...])


def _tc_fin(p, hp, dis, b):
    return pl.pallas_call(
        _fin_body,
        grid=(GRID,),
        in_specs=[
            pl.BlockSpec((NC, BN, D), lambda i: (0, i, 0)),
            pl.BlockSpec((BN, D), lambda i: (i, 0)),
            pl.BlockSpec((BN, D), lambda i: (i, 0)),
            pl.BlockSpec((1, D), lambda i: (0, 0)),
        ],
        out_specs=pl.BlockSpec((BN, D), lambda i: (i, 0)),
        out_shape=jax.ShapeDtypeStruct((NPAD, D), jnp.float32),
    )(p, hp, dis, b.reshape(1, D))


def kernel(x, edge_index, W1, b1, g1, be1, W2, b2, g2, be2, W3, b3):
    src = edge_index[0].reshape(NW * NCHUNK, 1, CHUNK)
    dst = edge_index[1].reshape(NW * NCHUNK, 1, CHUNK)
    xp = jnp.pad(x, ((0, NPAD - N), (0, 0)))

    deg_kernel, agg_kernel = _sc_kernels()
    h1 = _tc_mm(xp, W1)
    degp = deg_kernel(dst)
    dis, hp1 = _dis_hp(degp, h1)
    p1 = agg_kernel(hp1, src, dst)
    hp2 = _tc_mid(p1, hp1, dis, b1, g1, be1, W2)
    p2 = agg_kernel(hp2, src, dst)
    hp3 = _tc_mid(p2, hp2, dis, b2, g2, be2, W3)
    p3 = agg_kernel(hp3, src, dst)
    out = _tc_fin(p3, hp3, dis, b3)
    return out[:N]


# speedup vs baseline: 33.9183x; 1.0574x over previous
import functools

import jax
import jax.numpy as jnp
from jax import lax
from jax.experimental import pallas as pl
from jax.experimental.pallas import tpu as pltpu
from jax.experimental.pallas import tpu_sc as plsc

N = 10000
E = 320000
D = 128
EPS = 1e-5

NC = 2
NS = 16
NW = NC * NS
EPW = E // NW
CHUNK = 80
NCHUNK = EPW // CHUNK
ISLOT = 6
UNROLL = 6
NPAD = 10240
STRIPE = NPAD // NS
ASTRIPE = N // NS
BN = 1024
GRID = NPAD // BN


def _deg_body(ei_hbm, out_hbm, idx_v, ones_v, zbuf_v, acc_sh, dsem):
    cid = lax.axis_index("c")
    sid = lax.axis_index("s")
    wid = sid * NC + cid

    for j in range(CHUNK // 16):
        ones_v[pl.ds(16 * j, 16)] = jnp.ones((16,), jnp.float32)

    def _zfill(i, carry):
        zbuf_v[pl.ds(i * 16, 16)] = jnp.zeros((16,), jnp.float32)
        return carry

    lax.fori_loop(0, STRIPE // 16, _zfill, 0)
    pltpu.sync_copy(zbuf_v, acc_sh.at[pl.ds(sid * STRIPE, STRIPE)])
    pltpu.sync_copy(ei_hbm.at[pl.ds((NW + wid) * NCHUNK, NCHUNK)], idx_v)
    plsc.subcore_barrier()

    pltpu.async_copy(ones_v, acc_sh.at[idx_v.at[0, 0]], dsem, add=True)
    pltpu.async_copy(ones_v, acc_sh.at[idx_v.at[1, 0]], dsem, add=True)

    def _body(j, carry):
        pltpu.make_async_copy(ones_v, acc_sh.at[idx_v.at[j, 0]],
                              dsem).wait()
        pltpu.async_copy(ones_v, acc_sh.at[idx_v.at[j + 2, 0]], dsem,
                         add=True)
        return carry

    lax.fori_loop(0, NCHUNK - 2, _body, 0)
    for j in (NCHUNK - 2, NCHUNK - 1):
        pltpu.make_async_copy(ones_v, acc_sh.at[idx_v.at[j, 0]],
                              dsem).wait()
    plsc.subcore_barrier()
    pltpu.sync_copy(acc_sh.at[pl.ds(sid * STRIPE, STRIPE)],
                    out_hbm.at[cid, pl.ds(sid * STRIPE, STRIPE)])


def _agg_body(h_hbm, ei_hbm, out_hbm,
              srcb_v, dstb_v, rows_v, acc_sh, semg, sems, semi):
    cid = lax.axis_index("c")
    sid = lax.axis_index("s")
    wid = sid * NC + cid

    def _zfill(i, carry):
        for k in range(D // 16):
            rows_v[0, i, pl.ds(16 * k, 16)] = jnp.zeros((16,), jnp.float32)
        return carry

    lax.fori_loop(0, CHUNK, _zfill, 0)
    for t in range(STRIPE // CHUNK):
        pltpu.async_copy(rows_v.at[0],
                         acc_sh.at[pl.ds(sid * STRIPE + CHUNK * t, CHUNK)],
                         semg.at[0])
    for t in range(STRIPE // CHUNK):
        pltpu.make_async_copy(
            rows_v.at[0],
            acc_sh.at[pl.ds(sid * STRIPE + CHUNK * t, CHUNK)],
            semg.at[0]).wait()
    plsc.subcore_barrier()

    sbase = wid * NCHUNK
    dbase = (NW + wid) * NCHUNK

    def _ifetch(j, c):
        pltpu.async_copy(ei_hbm.at[sbase + j], srcb_v.at[c % ISLOT], semi)
        pltpu.async_copy(ei_hbm.at[dbase + j], dstb_v.at[c % ISLOT], semi)

    def _iwait(j, c):
        pltpu.make_async_copy(ei_hbm.at[sbase + j],
                              srcb_v.at[c % ISLOT], semi).wait()
        pltpu.make_async_copy(ei_hbm.at[dbase + j],
                              dstb_v.at[c % ISLOT], semi).wait()

    def _gather(c, b):
        pltpu.async_copy(h_hbm.at[srcb_v.at[c % ISLOT, 0]], rows_v.at[b],
                         semg.at[b])

    def _gwait(c, b):
        pltpu.make_async_copy(h_hbm.at[srcb_v.at[c % ISLOT, 0]],
                              rows_v.at[b], semg.at[b]).wait()

    def _scat(c, b):
        pltpu.async_copy(rows_v.at[b], acc_sh.at[dstb_v.at[c % ISLOT, 0]],
                         sems.at[b], add=True)

    def _swait(c, b):
        pltpu.make_async_copy(rows_v.at[b],
                              acc_sh.at[dstb_v.at[c % ISLOT, 0]],
                              sems.at[b]).wait()

    def _step(j, c, first=False, ng=True, nf=True):
        b = c % 3
        bn = (c + 2) % 3
        if not first:
            _swait(c - 1, bn)
        if ng:
            _iwait(j + 2, c + 2)
            _gather(c + 2, bn)
        if nf:
            _ifetch(j + 4, c + 4)
        _gwait(c, b)
        _scat(c, b)

    for j in range(4):
        _ifetch(j, j)
    _iwait(0, 0)
    _gather(0, 0)
    _iwait(1, 1)
    _gather(1, 1)

    _step(0, 0, first=True)
    for c in range(1, UNROLL):
        _step(c, c)

    def _group(t, carry):
        for c in range(UNROLL):
            _step(UNROLL * t + c, c)
        return carry

    lax.fori_loop(1, (NCHUNK - 5) // UNROLL, _group, 0)
    for j in range(NCHUNK - 5, NCHUNK):
        _step(j, j % UNROLL, ng=(j + 2 < NCHUNK), nf=(j + 4 < NCHUNK))
    _swait((NCHUNK - 1) % UNROLL, (NCHUNK - 1) % 3)
    plsc.subcore_barrier()
    pltpu.sync_copy(acc_sh.at[pl.ds(sid * STRIPE, STRIPE)],
                    out_hbm.at[cid, pl.ds(sid * STRIPE, STRIPE)])


@functools.cache
def _sc_kernels():
    mesh = plsc.VectorSubcoreMesh(core_axis_name="c", subcore_axis_name="s",
                                  num_cores=NC, num_subcores=NS)
    deg = pl.kernel(
        _deg_body,
        out_type=jax.ShapeDtypeStruct((NC, NPAD), jnp.float32),
        mesh=mesh,
        scratch_types=[
            pltpu.VMEM((NCHUNK, 1, CHUNK), jnp.int32),
            pltpu.VMEM((CHUNK,), jnp.float32),
            pltpu.VMEM((STRIPE,), jnp.float32),
            pltpu.VMEM_SHARED((NPAD,), jnp.float32),
            pltpu.SemaphoreType.DMA,
        ],
    )
    agg = pl.kernel(
        _agg_body,
        out_type=jax.ShapeDtypeStruct((NC, NPAD, D), jnp.float32),
        mesh=mesh,
        scratch_types=[
            pltpu.VMEM((ISLOT, 1, CHUNK), jnp.int32),
            pltpu.VMEM((ISLOT, 1, CHUNK), jnp.int32),
            pltpu.VMEM((3, CHUNK, D), jnp.float32),
            pltpu.VMEM_SHARED((NPAD, D), jnp.float32),
            pltpu.SemaphoreType.DMA((3,)),
            pltpu.SemaphoreType.DMA((3,)),
            pltpu.SemaphoreType.DMA,
        ],
    )
    return deg, agg


def _mm_body(x_ref, w_ref, out_ref):
    out_ref[...] = jnp.dot(x_ref[...], w_ref[...],
                           preferred_element_type=jnp.float32)


def _tc_mm(x, w):
    return pl.pallas_call(
        _mm_body,
        grid=(GRID,),
        in_specs=[
            pl.BlockSpec((BN, D), lambda i: (i, 0)),
            pl.BlockSpec((D, D), lambda i: (0, 0)),
        ],
        out_specs=pl.BlockSpec((BN, D), lambda i: (i, 0)),
        out_shape=jax.ShapeDtypeStruct((N, D), jnp.float32),
    )(x, w)


def _dis_body(deg_ref, h_ref, dis_ref, hp_ref):
    d = deg_ref[...]
    s = d[0:1, :] + d[1:2, :] + 1.0
    col = jnp.transpose(s)
    dis = jnp.broadcast_to(lax.rsqrt(col), (BN, D))
    dis_ref[...] = dis
    hp_ref[...] = dis * h_ref[...]


def _dis_hp(degp, h1):
    return pl.pallas_call(
        _dis_body,
        grid=(GRID,),
        in_specs=[
            pl.BlockSpec((NC, BN), lambda i: (0, i)),
            pl.BlockSpec((BN, D), lambda i: (i, 0)),
        ],
        out_specs=[
            pl.BlockSpec((BN, D), lambda i: (i, 0)),
            pl.BlockSpec((BN, D), lambda i: (i, 0)),
        ],
        out_shape=[
            jax.ShapeDtypeStruct((N, D), jnp.float32),
            jax.ShapeDtypeStruct((N, D), jnp.float32),
        ],
    )(degp, h1)


def _mid_body(p_ref, hp_ref, dis_ref, b_ref, g_ref, be_ref, w_ref, out_ref):
    dis = dis_ref[...]
    a = dis * (p_ref[0] + p_ref[1] + hp_ref[...]) + b_ref[...]
    mu = jnp.mean(a, axis=-1, keepdims=True)
    var = jnp.mean((a - mu) ** 2, axis=-1, keepdims=True)
    r = jnp.maximum((a - mu) * lax.rsqrt(var + EPS) * g_ref[...] + be_ref[...],
                    0.0)
    out_ref[...] = dis * jnp.dot(r, w_ref[...],
                                 preferred_element_type=jnp.float32)


def _tc_mid(p, hp, dis, b, g, be, w):
    return pl.pallas_call(
        _mid_body,
        grid=(GRID,),
        in_specs=[
            pl.BlockSpec((NC, BN, D), lambda i: (0, i, 0)),
            pl.BlockSpec((BN, D), lambda i: (i, 0)),
            pl.BlockSpec((BN, D), lambda i: (i, 0)),
            pl.BlockSpec((1, D), lambda i: (0, 0)),
            pl.BlockSpec((1, D), lambda i: (0, 0)),
            pl.BlockSpec((1, D), lambda i: (0, 0)),
            pl.BlockSpec((D, D), lambda i: (0, 0)),
        ],
        out_specs=pl.BlockSpec((BN, D), lambda i: (i, 0)),
        out_shape=jax.ShapeDtypeStruct((N, D), jnp.float32),
    )(p, hp, dis, b.reshape(1, D), g.reshape(1, D), be.reshape(1, D), w)


def _fin_body(p_ref, hp_ref, dis_ref, b_ref, out_ref):
    out_ref[...] = (dis_ref[...] * (p_ref[0] + p_ref[1] + hp_ref[...])
                    + b_ref[...])


def _tc_fin(p, hp, dis, b):
    return pl.pallas_call(
        _fin_body,
        grid=(GRID,),
        in_specs=[
            pl.BlockSpec((NC, BN, D), lambda i: (0, i, 0)),
            pl.BlockSpec((BN, D), lambda i: (i, 0)),
            pl.BlockSpec((BN, D), lambda i: (i, 0)),
            pl.BlockSpec((1, D), lambda i: (0, 0)),
        ],
        out_specs=pl.BlockSpec((BN, D), lambda i: (i, 0)),
        out_shape=jax.ShapeDtypeStruct((N, D), jnp.float32),
    )(p, hp, dis, b.reshape(1, D))


def kernel(x, edge_index, W1, b1, g1, be1, W2, b2, g2, be2, W3, b3):
    ei = edge_index.reshape(2 * NW * NCHUNK, 1, CHUNK)

    deg_kernel, agg_kernel = _sc_kernels()
    h1 = _tc_mm(x, W1)
    degp = deg_kernel(ei)
    dis, hp1 = _dis_hp(degp, h1)
    p1 = agg_kernel(hp1, ei)
    hp2 = _tc_mid(p1, hp1, dis, b1, g1, be1, W2)
    p2 = agg_kernel(hp2, ei)
    hp3 = _tc_mid(p2, hp2, dis, b2, g2, be2, W3)
    p3 = agg_kernel(hp3, ei)
    return _tc_fin(p3, hp3, dis, b3)


# speedup vs baseline: 34.0514x; 1.0039x over previous
import functools

import jax
import jax.numpy as jnp
from jax import lax
from jax.experimental import pallas as pl
from jax.experimental.pallas import tpu as pltpu
from jax.experimental.pallas import tpu_sc as plsc

N = 10000
E = 320000
D = 128
EPS = 1e-5

NC = 2
NS = 16
NW = NC * NS
EPW = E // NW
CHUNK = 80
NCHUNK = EPW // CHUNK
ISLOT = 6
UNROLL = 6
NPAD = 10240
STRIPE = NPAD // NS
ASTRIPE = N // NS
BN = 1024
GRID = NPAD // BN


def _deg_body(ei_hbm, out_hbm, idx_v, ones_v, zbuf_v, acc_sh, dsem):
    cid = lax.axis_index("c")
    sid = lax.axis_index("s")
    wid = sid * NC + cid

    for j in range(CHUNK // 16):
        ones_v[pl.ds(16 * j, 16)] = jnp.ones((16,), jnp.float32)

    def _zfill(i, carry):
        zbuf_v[pl.ds(i * 16, 16)] = jnp.zeros((16,), jnp.float32)
        return carry

    lax.fori_loop(0, STRIPE // 16, _zfill, 0)
    pltpu.sync_copy(zbuf_v, acc_sh.at[pl.ds(sid * STRIPE, STRIPE)])
    dbase = (NW + wid) * NCHUNK

    def _iload(j, carry):
        pltpu.async_copy(ei_hbm.at[pl.ds((dbase + j) * CHUNK, CHUNK)],
                         idx_v.at[j], dsem)
        return carry

    def _iloadw(j, carry):
        pltpu.make_async_copy(ei_hbm.at[pl.ds((dbase + j) * CHUNK, CHUNK)],
                              idx_v.at[j], dsem).wait()
        return carry

    lax.fori_loop(0, NCHUNK, _iload, 0)
    lax.fori_loop(0, NCHUNK, _iloadw, 0)
    plsc.subcore_barrier()

    pltpu.async_copy(ones_v, acc_sh.at[idx_v.at[0]], dsem, add=True)
    pltpu.async_copy(ones_v, acc_sh.at[idx_v.at[1]], dsem, add=True)

    def _body(j, carry):
        pltpu.make_async_copy(ones_v, acc_sh.at[idx_v.at[j]], dsem).wait()
        pltpu.async_copy(ones_v, acc_sh.at[idx_v.at[j + 2]], dsem,
                         add=True)
        return carry

    lax.fori_loop(0, NCHUNK - 2, _body, 0)
    for j in (NCHUNK - 2, NCHUNK - 1):
        pltpu.make_async_copy(ones_v, acc_sh.at[idx_v.at[j]], dsem).wait()
    plsc.subcore_barrier()
    pltpu.sync_copy(acc_sh.at[pl.ds(sid * STRIPE, STRIPE)],
                    out_hbm.at[cid, pl.ds(sid * STRIPE, STRIPE)])


def _agg_body(h_hbm, ei_hbm, out_hbm,
              srcb_v, dstb_v, rows_v, acc_sh, semg, sems, semi):
    cid = lax.axis_index("c")
    sid = lax.axis_index("s")
    wid = sid * NC + cid

    def _zfill(i, carry):
        for k in range(D // 16):
            rows_v[0, i, pl.ds(16 * k, 16)] = jnp.zeros((16,), jnp.float32)
        return carry

    lax.fori_loop(0, CHUNK, _zfill, 0)
    for t in range(STRIPE // CHUNK):
        pltpu.async_copy(rows_v.at[0],
                         acc_sh.at[pl.ds(sid * STRIPE + CHUNK * t, CHUNK)],
                         semg.at[0])
    for t in range(STRIPE // CHUNK):
        pltpu.make_async_copy(
            rows_v.at[0],
            acc_sh.at[pl.ds(sid * STRIPE + CHUNK * t, CHUNK)],
            semg.at[0]).wait()
    plsc.subcore_barrier()

    sbase = wid * NCHUNK
    dbase = (NW + wid) * NCHUNK

    def _ifetch(j, c):
        pltpu.async_copy(ei_hbm.at[pl.ds((sbase + j) * CHUNK, CHUNK)],
                         srcb_v.at[c % ISLOT], semi)
        pltpu.async_copy(ei_hbm.at[pl.ds((dbase + j) * CHUNK, CHUNK)],
                         dstb_v.at[c % ISLOT], semi)

    def _iwait(j, c):
        pltpu.make_async_copy(ei_hbm.at[pl.ds((sbase + j) * CHUNK, CHUNK)],
                              srcb_v.at[c % ISLOT], semi).wait()
        pltpu.make_async_copy(ei_hbm.at[pl.ds((dbase + j) * CHUNK, CHUNK)],
                              dstb_v.at[c % ISLOT], semi).wait()

    def _gather(c, b):
        pltpu.async_copy(h_hbm.at[srcb_v.at[c % ISLOT]], rows_v.at[b],
                         semg.at[b])

    def _gwait(c, b):
        pltpu.make_async_copy(h_hbm.at[srcb_v.at[c % ISLOT]],
                              rows_v.at[b], semg.at[b]).wait()

    def _scat(c, b):
        pltpu.async_copy(rows_v.at[b], acc_sh.at[dstb_v.at[c % ISLOT]],
                         sems.at[b], add=True)

    def _swait(c, b):
        pltpu.make_async_copy(rows_v.at[b],
                              acc_sh.at[dstb_v.at[c % ISLOT]],
                              sems.at[b]).wait()

    def _step(j, c, first=False, ng=True, nf=True):
        b = c % 3
        bn = (c + 2) % 3
        if not first:
            _swait(c - 1, bn)
        if ng:
            _iwait(j + 2, c + 2)
            _gather(c + 2, bn)
        if nf:
            _ifetch(j + 4, c + 4)
        _gwait(c, b)
        _scat(c, b)

    for j in range(4):
        _ifetch(j, j)
    _iwait(0, 0)
    _gather(0, 0)
    _iwait(1, 1)
    _gather(1, 1)

    _step(0, 0, first=True)
    for c in range(1, UNROLL):
        _step(c, c)

    def _group(t, carry):
        for c in range(UNROLL):
            _step(UNROLL * t + c, c)
        return carry

    lax.fori_loop(1, (NCHUNK - 5) // UNROLL, _group, 0)
    for j in range(NCHUNK - 5, NCHUNK):
        _step(j, j % UNROLL, ng=(j + 2 < NCHUNK), nf=(j + 4 < NCHUNK))
    _swait((NCHUNK - 1) % UNROLL, (NCHUNK - 1) % 3)
    plsc.subcore_barrier()
    pltpu.sync_copy(acc_sh.at[pl.ds(sid * STRIPE, STRIPE)],
                    out_hbm.at[cid, pl.ds(sid * STRIPE, STRIPE)])


@functools.cache
def _sc_kernels():
    mesh = plsc.VectorSubcoreMesh(core_axis_name="c", subcore_axis_name="s",
                                  num_cores=NC, num_subcores=NS)
    deg = pl.kernel(
        _deg_body,
        out_type=jax.ShapeDtypeStruct((NC, NPAD), jnp.float32),
        mesh=mesh,
        scratch_types=[
            pltpu.VMEM((NCHUNK, CHUNK), jnp.int32),
            pltpu.VMEM((CHUNK,), jnp.float32),
            pltpu.VMEM((STRIPE,), jnp.float32),
            pltpu.VMEM_SHARED((NPAD,), jnp.float32),
            pltpu.SemaphoreType.DMA,
        ],
    )
    agg = pl.kernel(
        _agg_body,
        out_type=jax.ShapeDtypeStruct((NC, NPAD, D), jnp.float32),
        mesh=mesh,
        scratch_types=[
            pltpu.VMEM((ISLOT, CHUNK), jnp.int32),
            pltpu.VMEM((ISLOT, CHUNK), jnp.int32),
            pltpu.VMEM((3, CHUNK, D), jnp.float32),
            pltpu.VMEM_SHARED((NPAD, D), jnp.float32),
            pltpu.SemaphoreType.DMA((3,)),
            pltpu.SemaphoreType.DMA((3,)),
            pltpu.SemaphoreType.DMA,
        ],
    )
    return deg, agg


def _mm_body(x_ref, w_ref, out_ref):
    out_ref[...] = jnp.dot(x_ref[...], w_ref[...],
                           preferred_element_type=jnp.float32)


def _tc_mm(x, w):
    return pl.pallas_call(
        _mm_body,
        grid=(GRID,),
        in_specs=[
            pl.BlockSpec((BN, D), lambda i: (i, 0)),
            pl.BlockSpec((D, D), lambda i: (0, 0)),
        ],
        out_specs=pl.BlockSpec((BN, D), lambda i: (i, 0)),
        out_shape=jax.ShapeDtypeStruct((N, D), jnp.float32),
    )(x, w)


def _dis_body(deg_ref, h_ref, dis_ref, hp_ref):
    d = deg_ref[...]
    s = d[0:1, :] + d[1:2, :] + 1.0
    col = jnp.transpose(s)
    dis = lax.rsqrt(col)
    dis_ref[...] = dis
    hp_ref[...] = dis * h_ref[...]


def _dis_hp(degp, h1):
    return pl.pallas_call(
        _dis_body,
        grid=(GRID,),
        in_specs=[
            pl.BlockSpec((NC, BN), lambda i: (0, i)),
            pl.BlockSpec((BN, D), lambda i: (i, 0)),
        ],
        out_specs=[
            pl.BlockSpec((BN, 1), lambda i: (i, 0)),
            pl.BlockSpec((BN, D), lambda i: (i, 0)),
        ],
        out_shape=[
            jax.ShapeDtypeStruct((N, 1), jnp.float32),
            jax.ShapeDtypeStruct((N, D), jnp.float32),
        ],
    )(degp, h1)


def _mid_body(p_ref, hp_ref, dis_ref, b_ref, g_ref, be_ref, w_ref, out_ref):
    dis = dis_ref[...]
    a = dis * (p_ref[0] + p_ref[1] + hp_ref[...]) + b_ref[...]
    mu = jnp.mean(a, axis=-1, keepdims=True)
    var = jnp.mean((a - mu) ** 2, axis=-1, keepdims=True)
    r = jnp.maximum((a - mu) * lax.rsqrt(var + EPS) * g_ref[...] + be_ref[...],
                    0.0)
    out_ref[...] = dis * jnp.dot(r, w_ref[...],
                                 preferred_element_type=jnp.float32)


def _tc_mid(p, hp, dis, b, g, be, w):
    return pl.pallas_call(
        _mid_body,
        grid=(GRID,),
        in_specs=[
            pl.BlockSpec((NC, BN, D), lambda i: (0, i, 0)),
            pl.BlockSpec((BN, D), lambda i: (i, 0)),
            pl.BlockSpec((BN, 1), lambda i: (i, 0)),
            pl.BlockSpec((1, D), lambda i: (0, 0)),
            pl.BlockSpec((1, D), lambda i: (0, 0)),
            pl.BlockSpec((1, D), lambda i: (0, 0)),
            pl.BlockSpec((D, D), lambda i: (0, 0)),
        ],
        out_specs=pl.BlockSpec((BN, D), lambda i: (i, 0)),
        out_shape=jax.ShapeDtypeStruct((N, D), jnp.float32),
    )(p, hp, dis, b.reshape(1, D), g.reshape(1, D), be.reshape(1, D), w)


def _fin_body(p_ref, hp_ref, dis_ref, b_ref, out_ref):
    out_ref[...] = (dis_ref[...] * (p_ref[0] + p_ref[1] + hp_ref[...])
                    + b_ref[...])


def _tc_fin(p, hp, dis, b):
    return pl.pallas_call(
        _fin_body,
        grid=(GRID,),
        in_specs=[
            pl.BlockSpec((NC, BN, D), lambda i: (0, i, 0)),
            pl.BlockSpec((BN, D), lambda i: (i, 0)),
            pl.BlockSpec((BN, 1), lambda i: (i, 0)),
            pl.BlockSpec((1, D), lambda i: (0, 0)),
        ],
        out_specs=pl.BlockSpec((BN, D), lambda i: (i, 0)),
        out_shape=jax.ShapeDtypeStruct((N, D), jnp.float32),
    )(p, hp, dis, b.reshape(1, D))


def kernel(x, edge_index, W1, b1, g1, be1, W2, b2, g2, be2, W3, b3):
    ei = edge_index.reshape(2 * E)

    deg_kernel, agg_kernel = _sc_kernels()
    h1 = _tc_mm(x, W1)
    degp = deg_kernel(ei)
    dis, hp1 = _dis_hp(degp, h1)
    p1 = agg_kernel(hp1, ei)
    hp2 = _tc_mid(p1, hp1, dis, b1, g1, be1, W2)
    p2 = agg_kernel(hp2, ei)
    hp3 = _tc_mid(p2, hp2, dis, b2, g2, be2, W3)
    p3 = agg_kernel(hp3, ei)
    return _tc_fin(p3, hp3, dis, b3)


# speedup vs baseline: 34.7931x; 1.0218x over previous
import functools

import jax
import jax.numpy as jnp
from jax import lax
from jax.experimental import pallas as pl
from jax.experimental.pallas import tpu as pltpu
from jax.experimental.pallas import tpu_sc as plsc

N = 10000
E = 320000
D = 128
EPS = 1e-5

NC = 2
NS = 16
NW = NC * NS
EPW = E // NW
CHUNK = 80
NCHUNK = EPW // CHUNK
ISLOT = 6
UNROLL = 6
NPAD = 10240
STRIPE = NPAD // NS
ASTRIPE = N // NS
BN = 2048
GRID = NPAD // BN


def _deg_body(ei_hbm, out_hbm, idx_v, ones_v, zbuf_v, acc_sh, dsem):
    cid = lax.axis_index("c")
    sid = lax.axis_index("s")
    wid = sid * NC + cid

    for j in range(CHUNK // 16):
        ones_v[pl.ds(16 * j, 16)] = jnp.ones((16,), jnp.float32)

    def _zfill(i, carry):
        zbuf_v[pl.ds(i * 16, 16)] = jnp.zeros((16,), jnp.float32)
        return carry

    lax.fori_loop(0, STRIPE // 16, _zfill, 0)
    pltpu.sync_copy(zbuf_v, acc_sh.at[pl.ds(sid * STRIPE, STRIPE)])
    dbase = (NW + wid) * NCHUNK

    def _iload(j, carry):
        pltpu.async_copy(ei_hbm.at[pl.ds((dbase + j) * CHUNK, CHUNK)],
                         idx_v.at[j], dsem)
        return carry

    def _iloadw(j, carry):
        pltpu.make_async_copy(ei_hbm.at[pl.ds((dbase + j) * CHUNK, CHUNK)],
                              idx_v.at[j], dsem).wait()
        return carry

    lax.fori_loop(0, NCHUNK, _iload, 0)
    lax.fori_loop(0, NCHUNK, _iloadw, 0)
    plsc.subcore_barrier()

    pltpu.async_copy(ones_v, acc_sh.at[idx_v.at[0]], dsem, add=True)
    pltpu.async_copy(ones_v, acc_sh.at[idx_v.at[1]], dsem, add=True)

    def _body(j, carry):
        pltpu.make_async_copy(ones_v, acc_sh.at[idx_v.at[j]], dsem).wait()
        pltpu.async_copy(ones_v, acc_sh.at[idx_v.at[j + 2]], dsem,
                         add=True)
        return carry

    lax.fori_loop(0, NCHUNK - 2, _body, 0)
    for j in (NCHUNK - 2, NCHUNK - 1):
        pltpu.make_async_copy(ones_v, acc_sh.at[idx_v.at[j]], dsem).wait()
    plsc.subcore_barrier()
    pltpu.sync_copy(acc_sh.at[pl.ds(sid * STRIPE, STRIPE)],
                    out_hbm.at[cid, pl.ds(sid * STRIPE, STRIPE)])


def _agg_body(h_hbm, ei_hbm, out_hbm,
              srcb_v, dstb_v, rows_v, acc_sh, semg, sems, semi):
    cid = lax.axis_index("c")
    sid = lax.axis_index("s")
    wid = sid * NC + cid

    def _zfill(i, carry):
        for k in range(D // 16):
            rows_v[0, i, pl.ds(16 * k, 16)] = jnp.zeros((16,), jnp.float32)
        return carry

    lax.fori_loop(0, CHUNK, _zfill, 0)
    for t in range(STRIPE // CHUNK):
        pltpu.async_copy(rows_v.at[0],
                         acc_sh.at[pl.ds(sid * STRIPE + CHUNK * t, CHUNK)],
                         semg.at[0])
    for t in range(STRIPE // CHUNK):
        pltpu.make_async_copy(
            rows_v.at[0],
            acc_sh.at[pl.ds(sid * STRIPE + CHUNK * t, CHUNK)],
            semg.at[0]).wait()
    plsc.subcore_barrier()

    sbase = wid * NCHUNK
    dbase = (NW + wid) * NCHUNK

    def _ifetch(j, c):
        pltpu.async_copy(ei_hbm.at[pl.ds((sbase + j) * CHUNK, CHUNK)],
                         srcb_v.at[c % ISLOT], semi)
        pltpu.async_copy(ei_hbm.at[pl.ds((dbase + j) * CHUNK, CHUNK)],
                         dstb_v.at[c % ISLOT], semi)

    def _iwait(j, c):
        pltpu.make_async_copy(ei_hbm.at[pl.ds((sbase + j) * CHUNK, CHUNK)],
                              srcb_v.at[c % ISLOT], semi).wait()
        pltpu.make_async_copy(ei_hbm.at[pl.ds((dbase + j) * CHUNK, CHUNK)],
                              dstb_v.at[c % ISLOT], semi).wait()

    def _gather(c, b):
        pltpu.async_copy(h_hbm.at[srcb_v.at[c % ISLOT]], rows_v.at[b],
                         semg.at[b])

    def _gwait(c, b):
        pltpu.make_async_copy(h_hbm.at[srcb_v.at[c % ISLOT]],
                              rows_v.at[b], semg.at[b]).wait()

    def _scat(c, b):
        pltpu.async_copy(rows_v.at[b], acc_sh.at[dstb_v.at[c % ISLOT]],
                         sems.at[b], add=True)

    def _swait(c, b):
        pltpu.make_async_copy(rows_v.at[b],
                              acc_sh.at[dstb_v.at[c % ISLOT]],
                              sems.at[b]).wait()

    def _step(j, c, first=False, ng=True, nf=True):
        b = c % 3
        bn = (c + 2) % 3
        if not first:
            _swait(c - 1, bn)
        if ng:
            _iwait(j + 2, c + 2)
            _gather(c + 2, bn)
        if nf:
            _ifetch(j + 4, c + 4)
        _gwait(c, b)
        _scat(c, b)

    for j in range(4):
        _ifetch(j, j)
    _iwait(0, 0)
    _gather(0, 0)
    _iwait(1, 1)
    _gather(1, 1)

    _step(0, 0, first=True)
    for c in range(1, UNROLL):
        _step(c, c)

    def _group(t, carry):
        for c in range(UNROLL):
            _step(UNROLL * t + c, c)
        return carry

    lax.fori_loop(1, (NCHUNK - 5) // UNROLL, _group, 0)
    for j in range(NCHUNK - 5, NCHUNK):
        _step(j, j % UNROLL, ng=(j + 2 < NCHUNK), nf=(j + 4 < NCHUNK))
    _swait((NCHUNK - 1) % UNROLL, (NCHUNK - 1) % 3)
    plsc.subcore_barrier()
    pltpu.sync_copy(acc_sh.at[pl.ds(sid * STRIPE, STRIPE)],
                    out_hbm.at[cid, pl.ds(sid * STRIPE, STRIPE)])


@functools.cache
def _sc_kernels():
    mesh = plsc.VectorSubcoreMesh(core_axis_name="c", subcore_axis_name="s",
                                  num_cores=NC, num_subcores=NS)
    deg = pl.kernel(
        _deg_body,
        out_type=jax.ShapeDtypeStruct((NC, NPAD), jnp.float32),
        mesh=mesh,
        scratch_types=[
            pltpu.VMEM((NCHUNK, CHUNK), jnp.int32),
            pltpu.VMEM((CHUNK,), jnp.float32),
            pltpu.VMEM((STRIPE,), jnp.float32),
            pltpu.VMEM_SHARED((NPAD,), jnp.float32),
            pltpu.SemaphoreType.DMA,
        ],
    )
    agg = pl.kernel(
        _agg_body,
        out_type=jax.ShapeDtypeStruct((NC, NPAD, D), jnp.float32),
        mesh=mesh,
        scratch_types=[
            pltpu.VMEM((ISLOT, CHUNK), jnp.int32),
            pltpu.VMEM((ISLOT, CHUNK), jnp.int32),
            pltpu.VMEM((3, CHUNK, D), jnp.float32),
            pltpu.VMEM_SHARED((NPAD, D), jnp.float32),
            pltpu.SemaphoreType.DMA((3,)),
            pltpu.SemaphoreType.DMA((3,)),
            pltpu.SemaphoreType.DMA,
        ],
    )
    return deg, agg


def _mm_body(x_ref, w_ref, out_ref):
    out_ref[...] = jnp.dot(x_ref[...], w_ref[...],
                           preferred_element_type=jnp.float32)


def _tc_mm(x, w):
    return pl.pallas_call(
        _mm_body,
        grid=(GRID,),
        in_specs=[
            pl.BlockSpec((BN, D), lambda i: (i, 0)),
            pl.BlockSpec((D, D), lambda i: (0, 0)),
        ],
        out_specs=pl.BlockSpec((BN, D), lambda i: (i, 0)),
        out_shape=jax.ShapeDtypeStruct((N, D), jnp.float32),
    )(x, w)


def _dis_body(deg_ref, h_ref, dis_ref, hp_ref):
    d = deg_ref[...]
    s = d[0:1, :] + d[1:2, :] + 1.0
    col = jnp.transpose(s)
    dis = lax.rsqrt(col)
    dis_ref[...] = dis
    hp_ref[...] = dis * h_ref[...]


def _dis_hp(degp, h1):
    return pl.pallas_call(
        _dis_body,
        grid=(GRID,),
        in_specs=[
            pl.BlockSpec((NC, BN), lambda i: (0, i)),
            pl.BlockSpec((BN, D), lambda i: (i, 0)),
        ],
        out_specs=[
            pl.BlockSpec((BN, 1), lambda i: (i, 0)),
            pl.BlockSpec((BN, D), lambda i: (i, 0)),
        ],
        out_shape=[
            jax.ShapeDtypeStruct((N, 1), jnp.float32),
            jax.ShapeDtypeStruct((N, D), jnp.float32),
        ],
    )(degp, h1)


def _mid_body(p_ref, hp_ref, dis_ref, b_ref, g_ref, be_ref, w_ref, out_ref):
    dis = dis_ref[...]
    a = dis * (p_ref[0] + p_ref[1] + hp_ref[...]) + b_ref[...]
    mu = jnp.mean(a, axis=-1, keepdims=True)
    var = jnp.mean((a - mu) ** 2, axis=-1, keepdims=True)
    r = jnp.maximum((a - mu) * lax.rsqrt(var + EPS) * g_ref[...] + be_ref[...],
                    0.0)
    out_ref[...] = dis * jnp.dot(r, w_ref[...],
                                 preferred_element_type=jnp.float32)


def _tc_mid(p, hp, dis, b, g, be, w):
    return pl.pallas_call(
        _mid_body,
        grid=(GRID,),
        in_specs=[
            pl.BlockSpec((NC, BN, D), lambda i: (0, i, 0)),
            pl.BlockSpec((BN, D), lambda i: (i, 0)),
            pl.BlockSpec((BN, 1), lambda i: (i, 0)),
            pl.BlockSpec((1, D), lambda i: (0, 0)),
            pl.BlockSpec((1, D), lambda i: (0, 0)),
            pl.BlockSpec((1, D), lambda i: (0, 0)),
            pl.BlockSpec((D, D), lambda i: (0, 0)),
        ],
        out_specs=pl.BlockSpec((BN, D), lambda i: (i, 0)),
        out_shape=jax.ShapeDtypeStruct((N, D), jnp.float32),
    )(p, hp, dis, b.reshape(1, D), g.reshape(1, D), be.reshape(1, D), w)


def _fin_body(p_ref, hp_ref, dis_ref, b_ref, out_ref):
    out_ref[...] = (dis_ref[...] * (p_ref[0] + p_ref[1] + hp_ref[...])
                    + b_ref[...])


def _tc_fin(p, hp, dis, b):
    return pl.pallas_call(
        _fin_body,
        grid=(GRID,),
        in_specs=[
            pl.BlockSpec((NC, BN, D), lambda i: (0, i, 0)),
            pl.BlockSpec((BN, D), lambda i: (i, 0)),
            pl.BlockSpec((BN, 1), lambda i: (i, 0)),
            pl.BlockSpec((1, D), lambda i: (0, 0)),
        ],
        out_specs=pl.BlockSpec((BN, D), lambda i: (i, 0)),
        out_shape=jax.ShapeDtypeStruct((N, D), jnp.float32),
    )(p, hp, dis, b.reshape(1, D))


def kernel(x, edge_index, W1, b1, g1, be1, W2, b2, g2, be2, W3, b3):
    ei = edge_index.reshape(2 * E)

    deg_kernel, agg_kernel = _sc_kernels()
    h1 = _tc_mm(x, W1)
    degp = deg_kernel(ei)
    dis, hp1 = _dis_hp(degp, h1)
    p1 = agg_kernel(hp1, ei)
    hp2 = _tc_mid(p1, hp1, dis, b1, g1, be1, W2)
    p2 = agg_kernel(hp2, ei)
    hp3 = _tc_mid(p2, hp2, dis, b2, g2, be2, W3)
    p3 = agg_kernel(hp3, ei)
    return _tc_fin(p3, hp3, dis, b3)
